# baseline (speedup 1.0000x reference)
import functools

import jax
import jax.numpy as jnp
from jax import lax
from jax.experimental import pallas as pl
from jax.experimental.pallas import tpu as pltpu
from jax.experimental.pallas import tpu_sc as plsc

N = 10000
E = 320000
D = 128
DE = 16

NC = 2
NS = 16
NW = NC * NS
CHUNK = 125
E_PER_W = E // NW
KW = E_PER_W // CHUNK
NCHUNK = E // CHUNK
NPAD = 10240
ROWS_PER_TILE = NPAD // NS


def _proj_body(x_ref, w_ref, ps_ref, pd_ref):
    acc = jnp.dot(x_ref[...], w_ref[...], preferred_element_type=jnp.float32)
    ps_ref[...] = acc[:, :DE]
    pd_ref[...] = acc[:, DE:]


def _node_proj(node_attr, w_sd):
    blk = 1000
    return pl.pallas_call(
        _proj_body,
        grid=(N // blk,),
        in_specs=[
            pl.BlockSpec((blk, D), lambda i: (i, 0)),
            pl.BlockSpec((D, 2 * DE), lambda i: (0, 0)),
        ],
        out_specs=[
            pl.BlockSpec((blk, DE), lambda i: (i, 0)),
            pl.BlockSpec((blk, DE), lambda i: (i, 0)),
        ],
        out_shape=[
            jax.ShapeDtypeStruct((N, DE), jnp.float32),
            jax.ShapeDtypeStruct((N, DE), jnp.float32),
        ],
    )(node_attr, w_sd)


def _pe_body(ea_ref, w_ref, b_ref, out_ref):
    out_ref[...] = (
        jnp.dot(ea_ref[...], w_ref[...], preferred_element_type=jnp.float32)
        + b_ref[...]
    )


def _edge_proj(ea128, w_blk, be_tiled):
    blk = 2000
    rows = E // 8
    return pl.pallas_call(
        _pe_body,
        grid=(rows // blk,),
        in_specs=[
            pl.BlockSpec((blk, D), lambda i: (i, 0)),
            pl.BlockSpec((D, D), lambda i: (0, 0)),
            pl.BlockSpec((1, D), lambda i: (0, 0)),
        ],
        out_specs=pl.BlockSpec((blk, D), lambda i: (i, 0)),
        out_shape=jax.ShapeDtypeStruct((rows, D), jnp.float32),
    )(ea128, w_blk, be_tiled.reshape(1, D))


def _sc_body(psrc, pdst, pe2d, row2, col2, eout2d, agg_hbm,
             ridx_all, cidx_all, src_v, acc_v, eo_v, zbuf, agg_sh,
             gsem0, gsem1, psem0, psem1, asem0, asem1,
             osem0, osem1, ssem0, ssem1):
    c = lax.axis_index("c")
    s = lax.axis_index("s")
    wid = s * NC + c
    gsem = (gsem0, gsem1)
    psem = (psem0, psem1)
    asem = (asem0, asem1)
    osem = (osem0, osem1)
    ssem = (ssem0, ssem1)

    def zero_row(i, carry):
        zbuf[i, :] = jnp.zeros((DE,), jnp.float32)
        return carry

    lax.fori_loop(0, ROWS_PER_TILE, zero_row, 0)
    pltpu.sync_copy(zbuf, agg_sh.at[pl.ds(s * ROWS_PER_TILE, ROWS_PER_TILE)])

    pltpu.sync_copy(row2.at[pl.ds(wid * KW, KW)], ridx_all)
    pltpu.sync_copy(col2.at[pl.ds(wid * KW, KW)], cidx_all)
    plsc.subcore_barrier()

    ebase = wid * E_PER_W

    def issue_p(j, b):
        pltpu.async_copy(pe2d.at[pl.ds(ebase + j * CHUNK, CHUNK)], acc_v.at[b],
                         psem[b])
        pltpu.async_copy(psrc.at[ridx_all.at[j]], src_v.at[b], gsem[b])

    def wait_pe(j, b):
        pltpu.make_async_copy(pe2d.at[pl.ds(ebase + j * CHUNK, CHUNK)],
                              acc_v.at[b], psem[b]).wait()

    def issue_a(j, b):
        pltpu.async_copy(pdst.at[cidx_all.at[j]], acc_v.at[b], asem[b], add=True)

    def wait_ga(j, b):
        pltpu.make_async_copy(psrc.at[ridx_all.at[j]], src_v.at[b], gsem[b]).wait()
        pltpu.make_async_copy(pdst.at[cidx_all.at[j]], acc_v.at[b],
                              asem[b]).wait()

    def issue_stores(j, b):
        pltpu.async_copy(eo_v.at[b],
                         eout2d.at[pl.ds(ebase + j * CHUNK, CHUNK)], osem[b])
        pltpu.async_copy(eo_v.at[b], agg_sh.at[cidx_all.at[j]], ssem[b], add=True)

    def wait_stores(j, b):
        pltpu.make_async_copy(
            eo_v.at[b], eout2d.at[pl.ds(ebase + j * CHUNK, CHUNK)], osem[b]
        ).wait()
        pltpu.make_async_copy(eo_v.at[b], agg_sh.at[cidx_all.at[j]], ssem[b]).wait()

    issue_p(0, 0)
    wait_pe(0, 0)
    issue_a(0, 0)

    def outer(i, carry):
        for b in (0, 1):
            j = 2 * i + b

            @pl.when(j + 1 < KW)
            def _():
                issue_p(j + 1, 1 - b)

            wait_ga(j, b)

            @pl.when(j >= 2)
            def _():
                wait_stores(j - 2, b)

            def fuse_row(r, rc):
                eo_v[b, r, :] = jnp.maximum(src_v[b, r, :] + acc_v[b, r, :], 0.0)
                return rc

            lax.fori_loop(0, CHUNK, fuse_row, 0, unroll=4)
            issue_stores(j, b)

            @pl.when(j + 1 < KW)
            def _():
                wait_pe(j + 1, 1 - b)
                issue_a(j + 1, 1 - b)
        return carry

    lax.fori_loop(0, KW // 2, outer, 0)
    wait_stores(KW - 2, 0)
    wait_stores(KW - 1, 1)
    plsc.subcore_barrier()
    sl = pl.ds(s * ROWS_PER_TILE, ROWS_PER_TILE)
    pltpu.sync_copy(agg_sh.at[sl], agg_hbm.at[c].at[sl])


def _sc_edges(psrc, pdst, pe2d, row2, col2):
    mesh = plsc.VectorSubcoreMesh(core_axis_name="c", subcore_axis_name="s")
    f = pl.kernel(
        _sc_body,
        out_type=(
            jax.ShapeDtypeStruct((E, DE), jnp.float32),
            jax.ShapeDtypeStruct((NC, NPAD, DE), jnp.float32),
        ),
        mesh=mesh,
        compiler_params=pltpu.CompilerParams(use_tc_tiling_on_sc=False),
        scratch_types=[
            pltpu.VMEM((KW, CHUNK), jnp.int32),
            pltpu.VMEM((KW, CHUNK), jnp.int32),
            pltpu.VMEM((2, CHUNK, DE), jnp.float32),
            pltpu.VMEM((2, CHUNK, DE), jnp.float32),
            pltpu.VMEM((2, CHUNK, DE), jnp.float32),
            pltpu.VMEM((ROWS_PER_TILE, DE), jnp.float32),
            pltpu.VMEM_SHARED((NPAD, DE), jnp.float32),
        ] + [pltpu.SemaphoreType.DMA] * 10,
    )
    return f(psrc, pdst, pe2d, row2, col2)


def _vout_body(x_ref, agg_ref, w1_ref, w2_ref, b_ref, o_ref):
    a = agg_ref[0] + agg_ref[1]
    acc = jnp.dot(x_ref[...], w1_ref[...], preferred_element_type=jnp.float32)
    acc = acc + jnp.dot(a, w2_ref[...], preferred_element_type=jnp.float32)
    o_ref[...] = jnp.maximum(acc + b_ref[...], 0.0)


def _node_update(node_attr, agg, w1, w2, bv):
    blk = 1000
    return pl.pallas_call(
        _vout_body,
        grid=(N // blk,),
        in_specs=[
            pl.BlockSpec((blk, D), lambda i: (i, 0)),
            pl.BlockSpec((NC, blk, DE), lambda i: (0, i, 0)),
            pl.BlockSpec((D, D), lambda i: (0, 0)),
            pl.BlockSpec((DE, D), lambda i: (0, 0)),
            pl.BlockSpec((1, D), lambda i: (0, 0)),
        ],
        out_specs=pl.BlockSpec((blk, D), lambda i: (i, 0)),
        out_shape=jax.ShapeDtypeStruct((N, D), jnp.float32),
    )(node_attr, agg, w1, w2, bv.reshape(1, D))


def kernel(node_attr, connectivity, edge_attr, u, We, be, Wv, bv):
    row2 = connectivity[0].reshape(NCHUNK, CHUNK)
    col2 = connectivity[1].reshape(NCHUNK, CHUNK)
    w_sd = jnp.concatenate([We[:D], We[D:2 * D]], axis=1)
    w_blk = jnp.kron(jnp.eye(8, dtype=jnp.float32), We[2 * D:])
    be_tiled = jnp.tile(be, 8)

    psrc, pdst = _node_proj(node_attr, w_sd)
    pe128 = _edge_proj(edge_attr.reshape(E // 8, D), w_blk, be_tiled)
    e_out, agg = _sc_edges(psrc, pdst, pe128.reshape(E, DE), row2, col2)
    v_out = _node_update(node_attr, agg, Wv[:D], Wv[D:], bv)
    global_attr = jnp.zeros_like(u)
    return (v_out, e_out, global_attr)


# speedup vs baseline: 1.0359x; 1.0359x over previous
import functools

import jax
import jax.numpy as jnp
from jax import lax
from jax.experimental import pallas as pl
from jax.experimental.pallas import tpu as pltpu
from jax.experimental.pallas import tpu_sc as plsc

N = 10000
E = 320000
D = 128
DE = 16

NC = 2
NS = 16
NW = NC * NS
CHUNK = 125
NSLAB = 2
ES = E // NSLAB
E_PER_W = ES // NW
KW = E_PER_W // CHUNK
NCHUNK = ES // CHUNK
NPAD = 10240
ROWS_PER_TILE = NPAD // NS


def _proj_body(x_ref, w_ref, ps_ref, pd_ref):
    acc = jnp.dot(x_ref[...], w_ref[...], preferred_element_type=jnp.float32)
    ps_ref[...] = acc[:, :DE]
    pd_ref[...] = acc[:, DE:]


def _node_proj(node_attr, w_sd):
    blk = 1000
    return pl.pallas_call(
        _proj_body,
        grid=(N // blk,),
        in_specs=[
            pl.BlockSpec((blk, D), lambda i: (i, 0)),
            pl.BlockSpec((D, 2 * DE), lambda i: (0, 0)),
        ],
        out_specs=[
            pl.BlockSpec((blk, DE), lambda i: (i, 0)),
            pl.BlockSpec((blk, DE), lambda i: (i, 0)),
        ],
        out_shape=[
            jax.ShapeDtypeStruct((N, DE), jnp.float32),
            jax.ShapeDtypeStruct((N, DE), jnp.float32),
        ],
    )(node_attr, w_sd)


def _pe_body(ea_ref, w_ref, b_ref, out_ref):
    out_ref[...] = (
        jnp.dot(ea_ref[...], w_ref[...], preferred_element_type=jnp.float32)
        + b_ref[...]
    )


def _edge_proj(ea128, w_blk, be_tiled):
    blk = 2000
    rows = ea128.shape[0]
    return pl.pallas_call(
        _pe_body,
        grid=(rows // blk,),
        in_specs=[
            pl.BlockSpec((blk, D), lambda i: (i, 0)),
            pl.BlockSpec((D, D), lambda i: (0, 0)),
            pl.BlockSpec((1, D), lambda i: (0, 0)),
        ],
        out_specs=pl.BlockSpec((blk, D), lambda i: (i, 0)),
        out_shape=jax.ShapeDtypeStruct((rows, D), jnp.float32),
    )(ea128, w_blk, be_tiled.reshape(1, D))


def _sc_body(psrc, pdst, pe2d, row2, col2, eout2d, agg_hbm,
             ridx_all, cidx_all, src_v, dst_v, acc_v, eo_v, zbuf, agg_sh,
             gsem0, gsem1, psem0, psem1, asem0, asem1,
             osem0, osem1, ssem0, ssem1):
    c = lax.axis_index("c")
    s = lax.axis_index("s")
    wid = s * NC + c
    gsem = (gsem0, gsem1)
    psem = (psem0, psem1)
    asem = (asem0, asem1)
    osem = (osem0, osem1)
    ssem = (ssem0, ssem1)

    def zero_row(i, carry):
        zbuf[i, :] = jnp.zeros((DE,), jnp.float32)
        return carry

    lax.fori_loop(0, ROWS_PER_TILE, zero_row, 0)
    pltpu.sync_copy(zbuf, agg_sh.at[pl.ds(s * ROWS_PER_TILE, ROWS_PER_TILE)])

    pltpu.sync_copy(row2.at[pl.ds(wid * KW, KW)], ridx_all)
    pltpu.sync_copy(col2.at[pl.ds(wid * KW, KW)], cidx_all)
    plsc.subcore_barrier()

    ebase = wid * E_PER_W

    def issue_loads(j, b):
        pltpu.async_copy(pe2d.at[pl.ds(ebase + j * CHUNK, CHUNK)], acc_v.at[b],
                         psem[b])
        pltpu.async_copy(psrc.at[ridx_all.at[j]], src_v.at[b], gsem[b])
        pltpu.async_copy(pdst.at[cidx_all.at[j]], dst_v.at[b], asem[b])

    def wait_loads(j, b):
        pltpu.make_async_copy(pe2d.at[pl.ds(ebase + j * CHUNK, CHUNK)],
                              acc_v.at[b], psem[b]).wait()
        pltpu.make_async_copy(psrc.at[ridx_all.at[j]], src_v.at[b], gsem[b]).wait()
        pltpu.make_async_copy(pdst.at[cidx_all.at[j]], dst_v.at[b],
                              asem[b]).wait()

    def issue_stores(j, b):
        pltpu.async_copy(eo_v.at[b],
                         eout2d.at[pl.ds(ebase + j * CHUNK, CHUNK)], osem[b])
        pltpu.async_copy(eo_v.at[b], agg_sh.at[cidx_all.at[j]], ssem[b], add=True)

    def wait_stores(j, b):
        pltpu.make_async_copy(
            eo_v.at[b], eout2d.at[pl.ds(ebase + j * CHUNK, CHUNK)], osem[b]
        ).wait()
        pltpu.make_async_copy(eo_v.at[b], agg_sh.at[cidx_all.at[j]], ssem[b]).wait()

    issue_loads(0, 0)

    def outer(i, carry):
        for b in (0, 1):
            j = 2 * i + b

            @pl.when(j + 1 < KW)
            def _():
                issue_loads(j + 1, 1 - b)

            wait_loads(j, b)

            @pl.when(j >= 2)
            def _():
                wait_stores(j - 2, b)

            def fuse_row(r, rc):
                eo_v[b, r, :] = jnp.maximum(
                    acc_v[b, r, :] + src_v[b, r, :] + dst_v[b, r, :], 0.0
                )
                return rc

            lax.fori_loop(0, CHUNK, fuse_row, 0, unroll=4)
            issue_stores(j, b)
        return carry

    lax.fori_loop(0, KW // 2, outer, 0)
    wait_stores(KW - 2, 0)
    wait_stores(KW - 1, 1)
    plsc.subcore_barrier()
    sl = pl.ds(s * ROWS_PER_TILE, ROWS_PER_TILE)
    pltpu.sync_copy(agg_sh.at[sl], agg_hbm.at[c].at[sl])


def _sc_edges(psrc, pdst, pe2d, row2, col2):
    mesh = plsc.VectorSubcoreMesh(core_axis_name="c", subcore_axis_name="s")
    f = pl.kernel(
        _sc_body,
        out_type=(
            jax.ShapeDtypeStruct((ES, DE), jnp.float32),
            jax.ShapeDtypeStruct((NC, NPAD, DE), jnp.float32),
        ),
        mesh=mesh,
        compiler_params=pltpu.CompilerParams(use_tc_tiling_on_sc=False),
        scratch_types=[
            pltpu.VMEM((KW, CHUNK), jnp.int32),
            pltpu.VMEM((KW, CHUNK), jnp.int32),
            pltpu.VMEM((2, CHUNK, DE), jnp.float32),
            pltpu.VMEM((2, CHUNK, DE), jnp.float32),
            pltpu.VMEM((2, CHUNK, DE), jnp.float32),
            pltpu.VMEM((2, CHUNK, DE), jnp.float32),
            pltpu.VMEM((ROWS_PER_TILE, DE), jnp.float32),
            pltpu.VMEM_SHARED((NPAD, DE), jnp.float32),
        ] + [pltpu.SemaphoreType.DMA] * 10,
    )
    return f(psrc, pdst, pe2d, row2, col2)


def _vout_body(x_ref, agg0_ref, agg1_ref, w1_ref, w2_ref, b_ref, o_ref):
    a = agg0_ref[0] + agg0_ref[1] + agg1_ref[0] + agg1_ref[1]
    acc = jnp.dot(x_ref[...], w1_ref[...], preferred_element_type=jnp.float32)
    acc = acc + jnp.dot(a, w2_ref[...], preferred_element_type=jnp.float32)
    o_ref[...] = jnp.maximum(acc + b_ref[...], 0.0)


def _node_update(node_attr, agg0, agg1, w1, w2, bv):
    blk = 1000
    return pl.pallas_call(
        _vout_body,
        grid=(N // blk,),
        in_specs=[
            pl.BlockSpec((blk, D), lambda i: (i, 0)),
            pl.BlockSpec((NC, blk, DE), lambda i: (0, i, 0)),
            pl.BlockSpec((NC, blk, DE), lambda i: (0, i, 0)),
            pl.BlockSpec((D, D), lambda i: (0, 0)),
            pl.BlockSpec((DE, D), lambda i: (0, 0)),
            pl.BlockSpec((1, D), lambda i: (0, 0)),
        ],
        out_specs=pl.BlockSpec((blk, D), lambda i: (i, 0)),
        out_shape=jax.ShapeDtypeStruct((N, D), jnp.float32),
    )(node_attr, agg0, agg1, w1, w2, bv.reshape(1, D))


def kernel(node_attr, connectivity, edge_attr, u, We, be, Wv, bv):
    w_sd = jnp.concatenate([We[:D], We[D:2 * D]], axis=1)
    w_blk = jnp.kron(jnp.eye(8, dtype=jnp.float32), We[2 * D:])
    be_tiled = jnp.tile(be, 8)

    psrc, pdst = _node_proj(node_attr, w_sd)
    e_outs = []
    aggs = []
    for sidx in range(NSLAB):
        sl = slice(sidx * ES, (sidx + 1) * ES)
        row2 = connectivity[0][sl].reshape(NCHUNK, CHUNK)
        col2 = connectivity[1][sl].reshape(NCHUNK, CHUNK)
        pe128 = _edge_proj(edge_attr[sl].reshape(ES // 8, D), w_blk, be_tiled)
        e_out_s, agg_s = _sc_edges(psrc, pdst, pe128.reshape(ES, DE), row2, col2)
        e_outs.append(e_out_s)
        aggs.append(agg_s)
    e_out = jnp.concatenate(e_outs, axis=0)
    v_out = _node_update(node_attr, aggs[0], aggs[1], Wv[:D], Wv[D:], bv)
    global_attr = jnp.zeros_like(u)
    return (v_out, e_out, global_attr)


# speedup vs baseline: 1.1932x; 1.1518x over previous
import functools

import jax
import jax.numpy as jnp
from jax import lax
from jax.experimental import pallas as pl
from jax.experimental.pallas import tpu as pltpu
from jax.experimental.pallas import tpu_sc as plsc

N = 10000
E = 320000
D = 128
DE = 16

NC = 2
NS = 16
NW = NC * NS
CHUNK = 125
NSLAB = 2
ES = E // NSLAB
E_PER_W = ES // NW
KW = E_PER_W // CHUNK
NCHUNK = ES // CHUNK
NPAD = 10240
ROWS_PER_TILE = NPAD // NS


def _proj_body(x_ref, w_ref, ps_ref, pd_ref):
    acc = jnp.dot(x_ref[...], w_ref[...], preferred_element_type=jnp.float32)
    ps_ref[...] = acc[:, :DE]
    pd_ref[...] = acc[:, DE:]


def _node_proj(node_attr, w_sd):
    blk = 1000
    return pl.pallas_call(
        _proj_body,
        grid=(N // blk,),
        in_specs=[
            pl.BlockSpec((blk, D), lambda i: (i, 0)),
            pl.BlockSpec((D, 2 * DE), lambda i: (0, 0)),
        ],
        out_specs=[
            pl.BlockSpec((blk, DE), lambda i: (i, 0)),
            pl.BlockSpec((blk, DE), lambda i: (i, 0)),
        ],
        out_shape=[
            jax.ShapeDtypeStruct((N, DE), jnp.float32),
            jax.ShapeDtypeStruct((N, DE), jnp.float32),
        ],
    )(node_attr, w_sd)


def _pe_body(ea_ref, w_ref, b_ref, out_ref):
    out_ref[...] = (
        jnp.dot(ea_ref[...], w_ref[...], preferred_element_type=jnp.float32)
        + b_ref[...]
    )


def _edge_proj(ea128, w_blk, be_tiled):
    blk = 2000
    rows = ea128.shape[0]
    return pl.pallas_call(
        _pe_body,
        grid=(rows // blk,),
        in_specs=[
            pl.BlockSpec((blk, D), lambda i: (i, 0)),
            pl.BlockSpec((D, D), lambda i: (0, 0)),
            pl.BlockSpec((1, D), lambda i: (0, 0)),
        ],
        out_specs=pl.BlockSpec((blk, D), lambda i: (i, 0)),
        out_shape=jax.ShapeDtypeStruct((rows, D), jnp.float32),
    )(ea128, w_blk, be_tiled.reshape(1, D))


def _sc_body(sidx, psrc, pdst, pe2d, row2, col2, eout2d, agg_hbm,
             ridx_all, cidx_all, src_v, dst_v, acc_v, eo_v, zbuf, agg_sh,
             gsem0, gsem1, psem0, psem1, asem0, asem1,
             osem0, osem1, ssem0, ssem1):
    c = lax.axis_index("c")
    s = lax.axis_index("s")
    wid = s * NC + c
    gsem = (gsem0, gsem1)
    psem = (psem0, psem1)
    asem = (asem0, asem1)
    osem = (osem0, osem1)
    ssem = (ssem0, ssem1)

    def zero_row(i, carry):
        zbuf[i, :] = jnp.zeros((DE,), jnp.float32)
        return carry

    lax.fori_loop(0, ROWS_PER_TILE, zero_row, 0)
    pltpu.sync_copy(zbuf, agg_sh.at[pl.ds(s * ROWS_PER_TILE, ROWS_PER_TILE)])

    cbase = sidx * NCHUNK + wid * KW
    pltpu.sync_copy(row2.at[pl.ds(cbase, KW)], ridx_all)
    pltpu.sync_copy(col2.at[pl.ds(cbase, KW)], cidx_all)
    plsc.subcore_barrier()

    ebase = wid * E_PER_W
    gbase = sidx * ES + ebase

    def issue_loads(j, b):
        pltpu.async_copy(pe2d.at[pl.ds(gbase + j * CHUNK, CHUNK)], acc_v.at[b],
                         psem[b])
        pltpu.async_copy(psrc.at[ridx_all.at[j]], src_v.at[b], gsem[b])
        pltpu.async_copy(pdst.at[cidx_all.at[j]], dst_v.at[b], asem[b])

    def wait_loads(j, b):
        pltpu.make_async_copy(pe2d.at[pl.ds(gbase + j * CHUNK, CHUNK)],
                              acc_v.at[b], psem[b]).wait()
        pltpu.make_async_copy(psrc.at[ridx_all.at[j]], src_v.at[b], gsem[b]).wait()
        pltpu.make_async_copy(pdst.at[cidx_all.at[j]], dst_v.at[b],
                              asem[b]).wait()

    def issue_stores(j, b):
        pltpu.async_copy(eo_v.at[b],
                         eout2d.at[pl.ds(ebase + j * CHUNK, CHUNK)], osem[b])
        pltpu.async_copy(eo_v.at[b], agg_sh.at[cidx_all.at[j]], ssem[b], add=True)

    def wait_stores(j, b):
        pltpu.make_async_copy(
            eo_v.at[b], eout2d.at[pl.ds(ebase + j * CHUNK, CHUNK)], osem[b]
        ).wait()
        pltpu.make_async_copy(eo_v.at[b], agg_sh.at[cidx_all.at[j]], ssem[b]).wait()

    issue_loads(0, 0)

    def outer(i, carry):
        for b in (0, 1):
            j = 2 * i + b

            @pl.when(j + 1 < KW)
            def _():
                issue_loads(j + 1, 1 - b)

            wait_loads(j, b)

            @pl.when(j >= 2)
            def _():
                wait_stores(j - 2, b)

            def fuse_row(r, rc):
                eo_v[b, r, :] = jnp.maximum(
                    acc_v[b, r, :] + src_v[b, r, :] + dst_v[b, r, :], 0.0
                )
                return rc

            lax.fori_loop(0, CHUNK, fuse_row, 0, unroll=4)
            issue_stores(j, b)
        return carry

    lax.fori_loop(0, KW // 2, outer, 0)
    wait_stores(KW - 2, 0)
    wait_stores(KW - 1, 1)
    plsc.subcore_barrier()
    sl = pl.ds(s * ROWS_PER_TILE, ROWS_PER_TILE)
    pltpu.sync_copy(agg_sh.at[sl], agg_hbm.at[c].at[sl])


def _sc_edges(psrc, pdst, pe2d, row2, col2, sidx):
    mesh = plsc.VectorSubcoreMesh(core_axis_name="c", subcore_axis_name="s")
    f = pl.kernel(
        functools.partial(_sc_body, sidx),
        out_type=(
            jax.ShapeDtypeStruct((ES, DE), jnp.float32),
            jax.ShapeDtypeStruct((NC, NPAD, DE), jnp.float32),
        ),
        mesh=mesh,
        compiler_params=pltpu.CompilerParams(use_tc_tiling_on_sc=False),
        scratch_types=[
            pltpu.VMEM((KW, CHUNK), jnp.int32),
            pltpu.VMEM((KW, CHUNK), jnp.int32),
            pltpu.VMEM((2, CHUNK, DE), jnp.float32),
            pltpu.VMEM((2, CHUNK, DE), jnp.float32),
            pltpu.VMEM((2, CHUNK, DE), jnp.float32),
            pltpu.VMEM((2, CHUNK, DE), jnp.float32),
            pltpu.VMEM((ROWS_PER_TILE, DE), jnp.float32),
            pltpu.VMEM_SHARED((NPAD, DE), jnp.float32),
        ] + [pltpu.SemaphoreType.DMA] * 10,
    )
    return f(psrc, pdst, pe2d, row2, col2)


def _vout_body(x_ref, agg0_ref, agg1_ref, w1_ref, w2_ref, b_ref, o_ref):
    a = agg0_ref[0] + agg0_ref[1] + agg1_ref[0] + agg1_ref[1]
    acc = jnp.dot(x_ref[...], w1_ref[...], preferred_element_type=jnp.float32)
    acc = acc + jnp.dot(a, w2_ref[...], preferred_element_type=jnp.float32)
    o_ref[...] = jnp.maximum(acc + b_ref[...], 0.0)


def _node_update(node_attr, agg0, agg1, w1, w2, bv):
    blk = 1000
    return pl.pallas_call(
        _vout_body,
        grid=(N // blk,),
        in_specs=[
            pl.BlockSpec((blk, D), lambda i: (i, 0)),
            pl.BlockSpec((NC, blk, DE), lambda i: (0, i, 0)),
            pl.BlockSpec((NC, blk, DE), lambda i: (0, i, 0)),
            pl.BlockSpec((D, D), lambda i: (0, 0)),
            pl.BlockSpec((DE, D), lambda i: (0, 0)),
            pl.BlockSpec((1, D), lambda i: (0, 0)),
        ],
        out_specs=pl.BlockSpec((blk, D), lambda i: (i, 0)),
        out_shape=jax.ShapeDtypeStruct((N, D), jnp.float32),
    )(node_attr, agg0, agg1, w1, w2, bv.reshape(1, D))


def kernel(node_attr, connectivity, edge_attr, u, We, be, Wv, bv):
    w_sd = jnp.concatenate([We[:D], We[D:2 * D]], axis=1)
    w_blk = jnp.kron(jnp.eye(8, dtype=jnp.float32), We[2 * D:])
    be_tiled = jnp.tile(be, 8)

    psrc, pdst = _node_proj(node_attr, w_sd)
    row2 = connectivity[0].reshape(NSLAB * NCHUNK, CHUNK)
    col2 = connectivity[1].reshape(NSLAB * NCHUNK, CHUNK)
    pe128 = _edge_proj(edge_attr.reshape(E // 8, D), w_blk, be_tiled)
    pe2d = pe128.reshape(E, DE)
    e_outs = []
    aggs = []
    for sidx in range(NSLAB):
        e_out_s, agg_s = _sc_edges(psrc, pdst, pe2d, row2, col2, sidx)
        e_outs.append(e_out_s)
        aggs.append(agg_s)
    e_out = jnp.concatenate(e_outs, axis=0)
    v_out = _node_update(node_attr, aggs[0], aggs[1], Wv[:D], Wv[D:], bv)
    global_attr = jnp.zeros_like(u)
    return (v_out, e_out, global_attr)
